# baseline (speedup 1.0000x reference)
import jax
import jax.numpy as jnp
from jax import lax
from jax.experimental import pallas as pl
from jax.experimental.pallas import tpu as pltpu
from jax.experimental.pallas import tpu_sc as plsc

N = 10000
E = 320000
D = 128
L = 16
NC = 2
NS = 16
NW = NC * NS
CH = 128
NCH = 80
EPW = NCH * CH
EPAD = NW * EPW
ROWS_PER_SUB = 624
TAIL = N - NS * ROWS_PER_SUB
ZR = 16


def _sc_body(x_hbm, idx_hbm, vals_hbm, part_hbm,
             acc, ibuf, vbuf, rows, zbuf, g0, g1, s0, s1, i0, i1, i2, i3):
  gsem = [g0, g1]
  ssem = [s0, s1]
  isem = [i0, i1, i2, i3]
  cid = lax.axis_index("c")
  sid = lax.axis_index("s")
  wid = cid * NS + sid

  def wait_rows(sem):
    pltpu.make_async_copy(x_hbm.at[pl.ds(0, CH)], rows.at[0], sem).wait()

  def wait_idx(sem, q):
    pltpu.make_async_copy(idx_hbm.at[wid, 0], ibuf.at[q], sem).wait()
    pltpu.make_async_copy(vals_hbm.at[wid, 0], vbuf.at[q], sem).wait()

  def fill_idx(sem, q, c):
    pltpu.async_copy(idx_hbm.at[wid, c], ibuf.at[q], sem)
    pltpu.async_copy(vals_hbm.at[wid, c], vbuf.at[q], sem)

  for q in range(3):
    fill_idx(isem[q], q, q)
  wait_idx(isem[0], 0)
  pltpu.async_copy(x_hbm.at[ibuf.at[0, 0]], rows.at[0], gsem[0])

  def zrow(r, carry):
    for k in range(D // L):
      zbuf[r, pl.ds(k * L, L)] = jnp.zeros((L,), jnp.float32)
    return carry
  lax.fori_loop(0, ZR, zrow, 0)
  base = sid * ROWS_PER_SUB

  def zcopy(i, carry):
    pltpu.sync_copy(zbuf, acc.at[pl.ds(base + i * ZR, ZR)])
    return carry
  lax.fori_loop(0, ROWS_PER_SUB // ZR, zcopy, 0)

  @pl.when(sid == NS - 1)
  def _zero_tail():
    pltpu.sync_copy(zbuf.at[pl.ds(0, TAIL)], acc.at[pl.ds(NS * ROWS_PER_SUB, TAIL)])
  plsc.subcore_barrier()

  dnums = lax.GatherDimensionNumbers(
      offset_dims=(), collapsed_slice_dims=(0,), start_index_map=(0,))

  def outer(it, carry):
    cbase = it * 4
    for b in range(4):
      c = cbase + b
      p = b % 2
      wait_rows(gsem[p])

      def group(g, gcarry):
        vv = vbuf[b, pl.ds(g * L, L)]
        for j in range(L):
          e = g * L + j
          vj = lax.gather(vv, jnp.full((L, 1), j, jnp.int32), dnums,
                          slice_sizes=(1,),
                          mode=lax.GatherScatterMode.PROMISE_IN_BOUNDS)
          for k in range(D // L):
            sl = pl.ds(k * L, L)
            rows[p, e, sl] = rows[p, e, sl] * vj
        return gcarry
      lax.fori_loop(0, CH // L, group, 0)

      @pl.when(c < NCH - 1)
      def _scatter_async():
        pltpu.async_copy(rows.at[p], acc.at[ibuf.at[b, 1]], ssem[p], add=True)

      @pl.when(c == NCH - 1)
      def _scatter_last():
        pltpu.sync_copy(rows.at[p], acc.at[ibuf.at[b, 1]], add=True)

      @pl.when(c >= 1)
      def _retire_scatter():
        wait_rows(ssem[1 - p])

      @pl.when(c + 1 < NCH)
      def _next_gather():
        q1 = (b + 1) % 4
        wait_idx(isem[q1], q1)
        pltpu.async_copy(x_hbm.at[ibuf.at[q1, 0]], rows.at[1 - p], gsem[1 - p])

      @pl.when(c + 3 < NCH)
      def _refill_idx():
        q3 = (b + 3) % 4
        fill_idx(isem[q3], q3, c + 3)
    return carry
  lax.fori_loop(0, NCH // 4, outer, 0)

  plsc.subcore_barrier()
  pltpu.sync_copy(acc.at[pl.ds(base, ROWS_PER_SUB)],
                  part_hbm.at[cid, pl.ds(base, ROWS_PER_SUB)])

  @pl.when(sid == NS - 1)
  def _write_tail():
    pltpu.sync_copy(acc.at[pl.ds(NS * ROWS_PER_SUB, TAIL)],
                    part_hbm.at[cid, pl.ds(NS * ROWS_PER_SUB, TAIL)])


@jax.jit
def _sc_spmm(x, idx_p, vals_p):
  mesh = plsc.VectorSubcoreMesh(core_axis_name="c", subcore_axis_name="s")
  return pl.kernel(
      _sc_body,
      out_type=jax.ShapeDtypeStruct((NC, N, D), jnp.float32),
      mesh=mesh,
      scratch_types=[
          pltpu.VMEM_SHARED((N, D), jnp.float32),
          pltpu.VMEM((4, 2, CH), jnp.int32),
          pltpu.VMEM((4, CH), jnp.float32),
          pltpu.VMEM((2, CH, D), jnp.float32),
          pltpu.VMEM((ZR, D), jnp.float32),
          pltpu.SemaphoreType.DMA,
          pltpu.SemaphoreType.DMA,
          pltpu.SemaphoreType.DMA,
          pltpu.SemaphoreType.DMA,
          pltpu.SemaphoreType.DMA,
          pltpu.SemaphoreType.DMA,
          pltpu.SemaphoreType.DMA,
          pltpu.SemaphoreType.DMA,
      ],
  )(x, idx_p, vals_p)


def _add_body(p_ref, o_ref):
  o_ref[...] = p_ref[0] + p_ref[1]


@jax.jit
def _combine(partials):
  rb = 1000
  return pl.pallas_call(
      _add_body,
      grid=(N // rb,),
      in_specs=[pl.BlockSpec((NC, rb, D), lambda i: (0, i, 0))],
      out_specs=pl.BlockSpec((rb, D), lambda i: (i, 0)),
      out_shape=jax.ShapeDtypeStruct((N, D), jnp.float32),
  )(partials)


def kernel(t, x, edge_index, edge_vals):
  src = edge_index[0].astype(jnp.int32)
  dst = edge_index[1].astype(jnp.int32)
  vals = edge_vals.astype(jnp.float32)
  pad = EPAD - E
  src = jnp.pad(src, (0, pad)).reshape(NW, NCH, CH)
  dst = jnp.pad(dst, (0, pad)).reshape(NW, NCH, CH)
  vals_p = jnp.pad(vals, (0, pad)).reshape(NW, NCH, CH)
  idx_p = jnp.stack([src, dst], axis=2)
  partials = _sc_spmm(x, idx_p, vals_p)
  return _combine(partials)


# speedup vs baseline: 1.0842x; 1.0842x over previous
import jax
import jax.numpy as jnp
from jax import lax
from jax.experimental import pallas as pl
from jax.experimental.pallas import tpu as pltpu
from jax.experimental.pallas import tpu_sc as plsc

N = 10000
E = 320000
D = 128
L = 16
NC = 2
NS = 16
NW = NC * NS
CH = 128
NPH = 5
P = 16
NCH = NPH * P
EPW = NCH * CH
EPAD = NW * EPW
ROWS_PER_SUB = 624
TAIL = N - NS * ROWS_PER_SUB
ZR = 16


def _sc_body(x_hbm, src_hbm, dst_hbm, vals_hbm, part_hbm,
             acc, src_v, dst_v, vals_v, rows, zbuf, g0, g1, s0, s1):
  gsem = [g0, g1]
  ssem = [s0, s1]
  cid = lax.axis_index("c")
  sid = lax.axis_index("s")
  wid = cid * NS + sid

  def wait_rows(sem):
    pltpu.make_async_copy(x_hbm.at[pl.ds(0, CH)], rows.at[0], sem).wait()

  def zrow(r, carry):
    for k in range(D // L):
      zbuf[r, pl.ds(k * L, L)] = jnp.zeros((L,), jnp.float32)
    return carry
  lax.fori_loop(0, ZR, zrow, 0)
  base = sid * ROWS_PER_SUB

  def zcopy(i, carry):
    pltpu.sync_copy(zbuf, acc.at[pl.ds(base + i * ZR, ZR)])
    return carry
  lax.fori_loop(0, ROWS_PER_SUB // ZR, zcopy, 0)

  @pl.when(sid == NS - 1)
  def _zero_tail():
    pltpu.sync_copy(zbuf.at[pl.ds(0, TAIL)], acc.at[pl.ds(NS * ROWS_PER_SUB, TAIL)])
  plsc.subcore_barrier()

  dnums = lax.GatherDimensionNumbers(
      offset_dims=(), collapsed_slice_dims=(0,), start_index_map=(0,))

  for ph in range(NPH):
    pltpu.sync_copy(src_hbm.at[wid, pl.ds(ph * P, P)], src_v)
    pltpu.sync_copy(dst_hbm.at[wid, pl.ds(ph * P, P)], dst_v)
    pltpu.sync_copy(vals_hbm.at[wid, pl.ds(ph * P, P)], vals_v)

    pltpu.async_copy(x_hbm.at[src_v.at[0]], rows.at[0], gsem[0])

    def inner(it, carry):
      cbase = it * 2
      for b in range(2):
        lc = cbase + b
        p = b
        @pl.when(lc >= 1)
        def _retire_scatter():
          wait_rows(ssem[1 - p])

        @pl.when(lc + 1 < P)
        def _next_gather():
          pltpu.async_copy(x_hbm.at[src_v.at[lc + 1]], rows.at[1 - p],
                           gsem[1 - p])

        wait_rows(gsem[p])

        def group(g, gcarry):
          vv = vals_v[lc, pl.ds(g * L, L)]
          for j in range(L):
            e = g * L + j
            vj = lax.gather(vv, jnp.full((L, 1), j, jnp.int32), dnums,
                            slice_sizes=(1,),
                            mode=lax.GatherScatterMode.PROMISE_IN_BOUNDS)
            for k in range(D // L):
              sl = pl.ds(k * L, L)
              rows[p, e, sl] = rows[p, e, sl] * vj
          return gcarry
        lax.fori_loop(0, CH // L, group, 0)

        @pl.when(lc < P - 1)
        def _scatter_async():
          pltpu.async_copy(rows.at[p], acc.at[dst_v.at[lc]], ssem[p],
                           add=True)

        @pl.when(lc == P - 1)
        def _scatter_last():
          pltpu.sync_copy(rows.at[p], acc.at[dst_v.at[lc]], add=True)
      return carry
    lax.fori_loop(0, P // 2, inner, 0)

  plsc.subcore_barrier()
  pltpu.sync_copy(acc.at[pl.ds(base, ROWS_PER_SUB)],
                  part_hbm.at[cid, pl.ds(base, ROWS_PER_SUB)])

  @pl.when(sid == NS - 1)
  def _write_tail():
    pltpu.sync_copy(acc.at[pl.ds(NS * ROWS_PER_SUB, TAIL)],
                    part_hbm.at[cid, pl.ds(NS * ROWS_PER_SUB, TAIL)])


@jax.jit
def _sc_spmm(x, src_p, dst_p, vals_p):
  mesh = plsc.VectorSubcoreMesh(core_axis_name="c", subcore_axis_name="s")
  return pl.kernel(
      _sc_body,
      out_type=jax.ShapeDtypeStruct((NC, N, D), jnp.float32),
      mesh=mesh,
      scratch_types=[
          pltpu.VMEM_SHARED((N, D), jnp.float32),
          pltpu.VMEM((P, CH), jnp.int32),
          pltpu.VMEM((P, CH), jnp.int32),
          pltpu.VMEM((P, CH), jnp.float32),
          pltpu.VMEM((2, CH, D), jnp.float32),
          pltpu.VMEM((ZR, D), jnp.float32),
          pltpu.SemaphoreType.DMA,
          pltpu.SemaphoreType.DMA,
          pltpu.SemaphoreType.DMA,
          pltpu.SemaphoreType.DMA,
      ],
  )(x, src_p, dst_p, vals_p)


def _add_body(p_ref, o_ref):
  o_ref[...] = p_ref[0] + p_ref[1]


@jax.jit
def _combine(partials):
  rb = 1000
  return pl.pallas_call(
      _add_body,
      grid=(N // rb,),
      in_specs=[pl.BlockSpec((NC, rb, D), lambda i: (0, i, 0))],
      out_specs=pl.BlockSpec((rb, D), lambda i: (i, 0)),
      out_shape=jax.ShapeDtypeStruct((N, D), jnp.float32),
  )(partials)


def kernel(t, x, edge_index, edge_vals):
  src = edge_index[0].astype(jnp.int32)
  dst = edge_index[1].astype(jnp.int32)
  vals = edge_vals.astype(jnp.float32)
  pad = EPAD - E
  src = jnp.pad(src, (0, pad)).reshape(NW, NCH, CH)
  dst = jnp.pad(dst, (0, pad)).reshape(NW, NCH, CH)
  vals_p = jnp.pad(vals, (0, pad)).reshape(NW, NCH, CH)
  partials = _sc_spmm(x, src, dst, vals_p)
  return _combine(partials)
